# Initial kernel scaffold; baseline (speedup 1.0000x reference)
import jax
import jax.numpy as jnp
from jax.experimental import pallas as pl


def kernel(qf, gf, W1, W2):
    raise NotImplementedError("write your pallas kernel here")


# speedup vs baseline: 206.3124x; 206.3124x over previous
import functools

import jax
import jax.numpy as jnp
from jax import lax
from jax.experimental import pallas as pl

D = 256
KNN = 10


def _stage1_body(xt_ref, xb_ref, w1_ref, h1_ref, idx_ref, *, R, B, TB):
    t = pl.program_id(0)
    base = (t % TB) * R
    xt = xt_ref[...]
    xb = xb_ref[...]
    sq_t = jnp.sum(xt * xt, axis=1)
    sq_b = jnp.sum(xb * xb, axis=1)
    g = jnp.dot(xt, xb.T, preferred_element_type=jnp.float32)
    d2 = sq_t[:, None] + sq_b[None, :] - 2.0 * g
    row_iota = lax.broadcasted_iota(jnp.int32, (R, B), 0)
    col_iota = lax.broadcasted_iota(jnp.int32, (R, B), 1)
    d2 = jnp.where(col_iota == row_iota + base, d2 + 1e9, d2)
    adj = jnp.zeros((R, B), jnp.float32)
    js = []
    for _ in range(KNN):
        m = jnp.min(d2, axis=1)
        cand = jnp.where(d2 == m[:, None], col_iota, B)
        j = jnp.min(cand, axis=1)
        sel = col_iota == j[:, None]
        adj = adj + sel.astype(jnp.float32)
        d2 = jnp.where(sel, jnp.float32(1e30), d2)
        js.append(j)
    idx_ref[...] = jnp.stack(js, axis=1)
    agg = (jnp.dot(adj, xb, preferred_element_type=jnp.float32) + xt) / 11.0
    h1_ref[...] = jnp.maximum(
        jnp.dot(agg, w1_ref[...], preferred_element_type=jnp.float32), 0.0)


def _stage1(x, w1, B, R):
    N = x.shape[0]
    TB = B // R
    return pl.pallas_call(
        functools.partial(_stage1_body, R=R, B=B, TB=TB),
        grid=(N // R,),
        in_specs=[
            pl.BlockSpec((R, D), lambda t: (t, 0)),
            pl.BlockSpec((B, D), lambda t: (t // TB, 0)),
            pl.BlockSpec((D, D), lambda t: (0, 0)),
        ],
        out_specs=[
            pl.BlockSpec((R, D), lambda t: (t, 0)),
            pl.BlockSpec((R, KNN), lambda t: (t, 0)),
        ],
        out_shape=[
            jax.ShapeDtypeStruct((N, D), jnp.float32),
            jax.ShapeDtypeStruct((N, KNN), jnp.int32),
        ],
    )(x, x, w1)


def _stage2_body(ht_ref, hb_ref, idx_ref, w2_ref, hn1_ref, hn2_ref, *, R, B):
    ht = ht_ref[...]
    hb = hb_ref[...]
    idx = idx_ref[...]
    col_iota = lax.broadcasted_iota(jnp.int32, (R, B), 1)
    adj = jnp.zeros((R, B), jnp.float32)
    for k in range(KNN):
        adj = adj + (col_iota == idx[:, k][:, None]).astype(jnp.float32)
    agg = (jnp.dot(adj, hb, preferred_element_type=jnp.float32) + ht) / 11.0
    h2 = jnp.maximum(
        jnp.dot(agg, w2_ref[...], preferred_element_type=jnp.float32), 0.0)
    hn1_ref[...] = ht / (jnp.sqrt(jnp.sum(ht * ht, axis=1, keepdims=True)) + 1e-12)
    hn2_ref[...] = h2 / (jnp.sqrt(jnp.sum(h2 * h2, axis=1, keepdims=True)) + 1e-12)


def _stage2(h1, idx, w2, B, R):
    N = h1.shape[0]
    TB = B // R
    return pl.pallas_call(
        functools.partial(_stage2_body, R=R, B=B),
        grid=(N // R,),
        in_specs=[
            pl.BlockSpec((R, D), lambda t: (t, 0)),
            pl.BlockSpec((B, D), lambda t: (t // TB, 0)),
            pl.BlockSpec((R, KNN), lambda t: (t, 0)),
            pl.BlockSpec((D, D), lambda t: (0, 0)),
        ],
        out_specs=[
            pl.BlockSpec((R, D), lambda t: (t, 0)),
            pl.BlockSpec((R, D), lambda t: (t, 0)),
        ],
        out_shape=[
            jax.ShapeDtypeStruct((N, D), jnp.float32),
            jax.ShapeDtypeStruct((N, D), jnp.float32),
        ],
    )(h1, h1, idx, w2)


def _stage3_body(q1_ref, g1_ref, q2_ref, g2_ref, s0_ref, sl_ref):
    s0_ref[...] = jnp.dot(q1_ref[...], g1_ref[...].T,
                          preferred_element_type=jnp.float32)
    sl_ref[...] = jnp.dot(q2_ref[...], g2_ref[...].T,
                          preferred_element_type=jnp.float32)


def _stage3(qn1, gn1, qn2, gn2, TQ=512, TG=2048):
    NQ, NG = qn1.shape[0], gn1.shape[0]
    return pl.pallas_call(
        _stage3_body,
        grid=(NQ // TQ, NG // TG),
        in_specs=[
            pl.BlockSpec((TQ, D), lambda i, j: (i, 0)),
            pl.BlockSpec((TG, D), lambda i, j: (j, 0)),
            pl.BlockSpec((TQ, D), lambda i, j: (i, 0)),
            pl.BlockSpec((TG, D), lambda i, j: (j, 0)),
        ],
        out_specs=[
            pl.BlockSpec((TQ, TG), lambda i, j: (i, j)),
            pl.BlockSpec((TQ, TG), lambda i, j: (i, j)),
        ],
        out_shape=[
            jax.ShapeDtypeStruct((NQ, NG), jnp.float32),
            jax.ShapeDtypeStruct((NQ, NG), jnp.float32),
        ],
    )(qn1, gn1, qn2, gn2)


def kernel(qf, gf, W1, W2):
    qh1, qidx = _stage1(qf, W1, B=1024, R=256)
    gh1, gidx = _stage1(gf, W1, B=4096, R=256)
    qn1, qn2 = _stage2(qh1, qidx, W2, B=1024, R=256)
    gn1, gn2 = _stage2(gh1, gidx, W2, B=4096, R=256)
    return _stage3(qn1, gn1, qn2, gn2)
